# baseline (speedup 1.0000x reference)
import functools

import jax
import jax.numpy as jnp
from jax import lax
from jax.experimental import pallas as pl
from jax.experimental.pallas import tpu as pltpu
from jax.experimental.pallas import tpu_sc as plsc

VOCAB = 1000000
SEQ_LEN = 2048
EMBED_DIM = 64
BATCH = 4
TOTAL = BATCH * SEQ_LEN
NUM_WORKERS = 32
ROWS_PER_W = TOTAL // NUM_WORKERS
LANES = 16
N_GROUPS = ROWS_PER_W // LANES
BLK = 16


def _body(idx_hbm, tok_hbm, pos_hbm, out_hbm, idx_v, rows_v, pos_v, *sems):
    wid = lax.axis_index("s") * 2 + lax.axis_index("c")
    base = wid * ROWS_PER_W
    pos_base = lax.rem(base, SEQ_LEN)

    pltpu.sync_copy(idx_hbm.at[pl.ds(base, ROWS_PER_W)], idx_v)
    pltpu.sync_copy(pos_hbm.at[pl.ds(pos_base, ROWS_PER_W)], pos_v)

    for g in range(N_GROUPS):
        j0 = g * LANES
        idx16 = idx_v[pl.ds(j0, LANES)]
        for jj in range(LANES):
            i = idx16[jj]
            t = lax.shift_right_logical(i, 4)
            r = lax.bitwise_and(i, BLK - 1)
            pltpu.async_copy(tok_hbm.at[t, r], rows_v.at[j0 + jj], sems[g])

    for g in range(N_GROUPS):
        pltpu.make_async_copy(
            pos_hbm.at[pl.ds(0, LANES)],
            rows_v.at[pl.ds(g * LANES, LANES)], sems[g]
        ).wait()

    def grp_fn(g, carry):
        j0 = g * LANES
        for jj in range(LANES):
            for c in range(EMBED_DIM // LANES):
                sl = pl.ds(c * LANES, LANES)
                rows_v[j0 + jj, sl] = rows_v[j0 + jj, sl] + pos_v[j0 + jj, sl]
        return carry

    lax.fori_loop(0, N_GROUPS, grp_fn, 0)

    pltpu.sync_copy(rows_v, out_hbm.at[pl.ds(base, ROWS_PER_W)])


@jax.jit
def _run(idx, tok3, position_table):
    mesh = plsc.VectorSubcoreMesh(core_axis_name="c", subcore_axis_name="s")
    f = functools.partial(
        pl.kernel,
        out_type=jax.ShapeDtypeStruct((TOTAL, EMBED_DIM), jnp.float32),
        mesh=mesh,
        scratch_types=[
            pltpu.VMEM((ROWS_PER_W,), jnp.int32),
            pltpu.VMEM((ROWS_PER_W, EMBED_DIM), jnp.float32),
            pltpu.VMEM((ROWS_PER_W, EMBED_DIM), jnp.float32),
        ] + [pltpu.SemaphoreType.DMA] * N_GROUPS,
        compiler_params=pltpu.CompilerParams(needs_layout_passes=True),
    )(_body)
    return f(idx, tok3, position_table)


def kernel(inputs, token_table, position_table):
    idx = inputs.astype(jnp.int32).reshape(TOTAL)
    tok3 = token_table.reshape(VOCAB // BLK, BLK, EMBED_DIM)
    out = _run(idx, tok3, position_table)
    return out.reshape(BATCH, SEQ_LEN, EMBED_DIM)


# speedup vs baseline: 1.0058x; 1.0058x over previous
import functools

import jax
import jax.numpy as jnp
from jax import lax
from jax.experimental import pallas as pl
from jax.experimental.pallas import tpu as pltpu
from jax.experimental.pallas import tpu_sc as plsc

VOCAB = 1000000
SEQ_LEN = 2048
EMBED_DIM = 64
BATCH = 4
TOTAL = BATCH * SEQ_LEN
NUM_WORKERS = 32
ROWS_PER_W = TOTAL // NUM_WORKERS
LANES = 16
N_GROUPS = ROWS_PER_W // LANES
BLK = 16


def _body(idx_hbm, tok_hbm, pos_hbm, out_hbm, idx_v, rows_v, pos_v, *sems):
    wid = lax.axis_index("s") * 2 + lax.axis_index("c")
    base = wid * ROWS_PER_W
    pos_base = lax.rem(base, SEQ_LEN)

    pltpu.sync_copy(idx_hbm.at[pl.ds(base, ROWS_PER_W)], idx_v)
    pltpu.sync_copy(pos_hbm.at[pl.ds(pos_base, ROWS_PER_W)], pos_v)

    for g in range(N_GROUPS):
        j0 = g * LANES
        idx16 = idx_v[pl.ds(j0, LANES)]
        for jj in range(LANES):
            i = idx16[jj]
            t = lax.shift_right_logical(i, 4)
            r = lax.bitwise_and(i, BLK - 1)
            pltpu.async_copy(tok_hbm.at[t, r], rows_v.at[j0 + jj], sems[g])

    for g in range(N_GROUPS):
        pltpu.make_async_copy(
            pos_hbm.at[pl.ds(0, LANES)],
            rows_v.at[pl.ds(g * LANES, LANES)], sems[g]
        ).wait()

        def add_fn(jj, carry, g=g):
            j = g * LANES + jj
            for c in range(EMBED_DIM // LANES):
                sl = pl.ds(c * LANES, LANES)
                rows_v[j, sl] = rows_v[j, sl] + pos_v[j, sl]
            return carry

        lax.fori_loop(0, LANES, add_fn, 0)

    pltpu.sync_copy(rows_v, out_hbm.at[pl.ds(base, ROWS_PER_W)])


@jax.jit
def _run(idx, tok3, position_table):
    mesh = plsc.VectorSubcoreMesh(core_axis_name="c", subcore_axis_name="s")
    f = functools.partial(
        pl.kernel,
        out_type=jax.ShapeDtypeStruct((TOTAL, EMBED_DIM), jnp.float32),
        mesh=mesh,
        scratch_types=[
            pltpu.VMEM((ROWS_PER_W,), jnp.int32),
            pltpu.VMEM((ROWS_PER_W, EMBED_DIM), jnp.float32),
            pltpu.VMEM((ROWS_PER_W, EMBED_DIM), jnp.float32),
        ] + [pltpu.SemaphoreType.DMA] * N_GROUPS,
    )(_body)
    return f(idx, tok3, position_table)


def kernel(inputs, token_table, position_table):
    idx = inputs.astype(jnp.int32).reshape(TOTAL)
    tok3 = token_table.reshape(VOCAB // BLK, BLK, EMBED_DIM)
    out = _run(idx, tok3, position_table)
    return out.reshape(BATCH, SEQ_LEN, EMBED_DIM)


# speedup vs baseline: 2.0499x; 2.0381x over previous
import functools

import jax
import jax.numpy as jnp
from jax import lax
from jax.experimental import pallas as pl
from jax.experimental.pallas import tpu as pltpu
from jax.experimental.pallas import tpu_sc as plsc

VOCAB = 1000000
SEQ_LEN = 2048
EMBED_DIM = 64
BATCH = 4
TOTAL = BATCH * SEQ_LEN
NUM_WORKERS = 32
ROWS_PER_W = TOTAL // NUM_WORKERS
LANES = 16
BCOL = 128
DEPTH = 8
N_HALF = ROWS_PER_W // DEPTH


def _body(idx_hbm, tok_hbm, pos_hbm, out_hbm,
          idx_v, blocks_v, pos_v, out2_v, *sems):
    sem_out = sems[DEPTH:]
    wid = lax.axis_index("s") * 2 + lax.axis_index("c")
    base = wid * ROWS_PER_W
    pos_base = lax.rem(base, SEQ_LEN)

    pltpu.sync_copy(idx_hbm.at[pl.ds(base, ROWS_PER_W)],
                    idx_v.at[pl.ds(0, ROWS_PER_W)])
    pltpu.sync_copy(pos_hbm.at[pl.ds(pos_base, ROWS_PER_W)], pos_v)

    lanes_c = lax.iota(jnp.int32, LANES)

    def fire(i, slot):
        b = lax.shift_right_logical(i, 7)
        pltpu.async_copy(tok_hbm.at[:, pl.ds(b * BCOL, BCOL)],
                         blocks_v.at[slot], sems[slot])

    def extract(j, jj, i, buf):
        col = lax.bitwise_and(i, BCOL - 1)
        pltpu.make_async_copy(tok_hbm.at[:, pl.ds(0, BCOL)],
                              blocks_v.at[jj], sems[jj]).wait()
        colv = lax.broadcast(col, (LANES,))
        slotv = lax.broadcast(jnp.int32(jj), (LANES,))
        for c in range(EMBED_DIM // LANES):
            sl = pl.ds(c * LANES, LANES)
            vals = plsc.load_gather(
                blocks_v, [slotv, lanes_c + c * LANES, colv])
            out2_v[buf, jj, sl] = vals + pos_v[j, sl]

    def flush(h, buf):
        pltpu.async_copy(out2_v.at[buf],
                         out_hbm.at[pl.ds(base + h * DEPTH, DEPTH)],
                         sem_out[buf])

    def drain_flush(buf):
        pltpu.make_async_copy(out2_v.at[buf],
                              out_hbm.at[pl.ds(0, DEPTH)],
                              sem_out[buf]).wait()

    idx16 = idx_v[pl.ds(0, LANES)]
    for jj in range(DEPTH):
        fire(idx16[jj], jj)

    def run_half(h, buf):
        j0 = h * DEPTH
        cur16 = idx_v[pl.ds(j0, LANES)]
        nxt16 = idx_v[pl.ds(j0 + DEPTH, LANES)]
        for jj in range(DEPTH):
            extract(j0 + jj, jj, cur16[jj], buf)
            fire(nxt16[jj], jj)
        flush(h, buf)

    def pair_fn(h2, carry):
        @pl.when(h2 >= 1)
        def _():
            drain_flush(0)
        run_half(2 * h2, 0)

        @pl.when(h2 >= 1)
        def _():
            drain_flush(1)
        run_half(2 * h2 + 1, 1)
        return carry

    lax.fori_loop(0, (N_HALF - 2) // 2, pair_fn, 0)

    h = N_HALF - 2
    drain_flush(0)
    run_half(h, 0)

    h = N_HALF - 1
    j0 = h * DEPTH
    last16 = idx_v[pl.ds(j0, LANES)]
    drain_flush(1)
    for jj in range(DEPTH):
        extract(j0 + jj, jj, last16[jj], 1)
    flush(h, 1)
    drain_flush(0)
    drain_flush(1)


@jax.jit
def _run(idx, tok_t, position_table):
    mesh = plsc.VectorSubcoreMesh(core_axis_name="c", subcore_axis_name="s")
    f = functools.partial(
        pl.kernel,
        out_type=jax.ShapeDtypeStruct((TOTAL, EMBED_DIM), jnp.float32),
        mesh=mesh,
        scratch_types=[
            pltpu.VMEM((ROWS_PER_W + LANES,), jnp.int32),
            pltpu.VMEM((DEPTH, EMBED_DIM, BCOL), jnp.float32),
            pltpu.VMEM((ROWS_PER_W, EMBED_DIM), jnp.float32),
            pltpu.VMEM((2, DEPTH, EMBED_DIM), jnp.float32),
        ] + [pltpu.SemaphoreType.DMA] * (DEPTH + 2),
        compiler_params=pltpu.CompilerParams(needs_layout_passes=False),
    )(_body)
    return f(idx, tok_t, position_table)


def kernel(inputs, token_table, position_table):
    idx = inputs.astype(jnp.int32).reshape(TOTAL)
    tok_t = token_table.T
    out = _run(idx, tok_t, position_table)
    return out.reshape(BATCH, SEQ_LEN, EMBED_DIM)
